# baseline (speedup 1.0000x reference)
import functools

import jax
import jax.numpy as jnp
from jax import lax
from jax.experimental import pallas as pl
from jax.experimental.pallas import tpu as pltpu
from jax.experimental.pallas import tpu_sc as plsc

B = 130000
V = 1000000
H = 64
L = 16
NC, NS = 2, 16
NW = NC * NS
BP = 131072
BW = BP // NW
CH = 128
NCHUNK = BW // CH
NB = 2
NGRP = NCHUNK // NB
RW = BW // CH
CB = 512
VP = pl.cdiv(V, CB) * CB


def _tc_pack(wT, cT):
    def body(w_ref, c_ref, o_ref):
        o_ref[:, 0:H] = w_ref[...].T
        o_ref[:, H:2 * H] = c_ref[...].T

    return pl.pallas_call(
        body,
        grid=(VP // CB,),
        in_specs=[pl.BlockSpec((H, CB), lambda i: (0, i)),
                  pl.BlockSpec((H, CB), lambda i: (0, i))],
        out_specs=pl.BlockSpec((CB, 2 * H), lambda i: (i, 0)),
        out_shape=jax.ShapeDtypeStruct((VP, 2 * H), jnp.float32),
    )(wT, cT)


def _sc_dots(wrd2d, cntxt2d, comb):
    mesh = plsc.VectorSubcoreMesh(core_axis_name="c", subcore_axis_name="s")

    @functools.partial(
        pl.kernel,
        compiler_params=pltpu.CompilerParams(
            needs_layout_passes=False, use_tc_tiling_on_sc=True),
        out_type=jax.ShapeDtypeStruct((BP // CH, CH), jnp.float32),
        mesh=mesh,
        scratch_types=[
            pltpu.VMEM((NCHUNK, CH), jnp.int32),
            pltpu.VMEM((NCHUNK, CH), jnp.int32),
            [pltpu.VMEM((CH, 2 * H), jnp.float32)] * NB,
            [pltpu.VMEM((CH, 2 * H), jnp.float32)] * NB,
            pltpu.VMEM((NCHUNK, CH), jnp.float32),
            [pltpu.SemaphoreType.DMA] * NB,
        ],
    )
    def k(wrd_h, cx_h, comb_h, out_h,
          widx, cidx, wrows, crows, dots, gsems):
        wid = lax.axis_index("s") * NC + lax.axis_index("c")
        r0 = pl.multiple_of(wid * RW, RW)

        pltpu.sync_copy(wrd_h.at[pl.ds(r0, RW)], widx)
        pltpu.sync_copy(cx_h.at[pl.ds(r0, RW)], cidx)

        def chunk_start(g, b):
            def sub(s, _):
                off = pl.multiple_of(s * L, L)
                ivw = widx[g, pl.ds(off, L)]
                ivc = cidx[g, pl.ds(off, L)]
                for j in range(L):
                    pltpu.make_async_copy(
                        comb_h.at[pl.ds(ivw[j], 1)],
                        wrows[b].at[pl.ds(s * L + j, 1)], gsems[b]).start()
                    pltpu.make_async_copy(
                        comb_h.at[pl.ds(ivc[j], 1)],
                        crows[b].at[pl.ds(s * L + j, 1)], gsems[b]).start()
                return 0
            lax.fori_loop(0, CH // L, sub, 0)

        def chunk_wait(b):
            pltpu.make_async_copy(
                comb_h.at[pl.ds(0, CH)], wrows[b], gsems[b]).wait()
            pltpu.make_async_copy(
                comb_h.at[pl.ds(0, CH)], crows[b], gsems[b]).wait()

        for b in range(NB):
            chunk_start(b, b)

        lanes = lax.iota(jnp.int32, L)
        lastlane = lanes == (L - 1)
        gsplat_cache = {}

        def dotitem(wr, cr, i):
            parts = []
            for q in range(4):
                parts.append(wr[i, pl.ds(q * L, L)]
                             * cr[i, pl.ds(H + q * L, L)])
            t = (parts[0] + parts[1]) + (parts[2] + parts[3])
            c = plsc.cumsum(t)
            plsc.store_scatter(
                dots, [gsplat_cache[0], jnp.full((L,), i, jnp.int32)],
                c, mask=lastlane)

        def compute(g, b):
            gsplat_cache[0] = jnp.full((L,), g, jnp.int32)

            def body(it, _):
                i = it * 2
                dotitem(wrows[b], crows[b], i)
                dotitem(wrows[b], crows[b], i + 1)
                return 0
            lax.fori_loop(0, CH // 2, body, 0)

        def grp_body(grp, _):
            for b in range(NB):
                g = grp * NB + b
                chunk_wait(b)
                compute(g, b)

                @pl.when(grp < NGRP - 1)
                def _():
                    chunk_start(g + NB, b)
            return 0

        lax.fori_loop(0, NGRP, grp_body, 0)

        pltpu.sync_copy(dots, out_h.at[pl.ds(r0, RW)])

    return k(wrd2d, cntxt2d, comb)


def _tc_loss(dots2d, labels2d):
    def body(d_ref, l_ref, o_ref):
        x = d_ref[...] * l_ref[...]
        r = lax.broadcasted_iota(jnp.int32, x.shape, 0)
        c = lax.broadcasted_iota(jnp.int32, x.shape, 1)
        valid = (r * x.shape[1] + c) < B
        ls = jnp.where(valid, jax.nn.log_sigmoid(x), 0.0)
        o_ref[0, 0] = jnp.sum(ls) * (-1.0 / B)

    out = pl.pallas_call(
        body,
        out_shape=jax.ShapeDtypeStruct((1, 1), jnp.float32),
        out_specs=pl.BlockSpec(memory_space=pltpu.SMEM),
    )(dots2d, labels2d)
    return out[0, 0]


def kernel(wrd, cntxt, labels, word_emb, context_emb):
    pad = BP - B
    wrd_p = jnp.concatenate(
        [wrd.reshape(-1), jnp.zeros((pad,), jnp.int32)]).reshape(BP // CH, CH)
    cx_p = jnp.concatenate(
        [cntxt.reshape(-1), jnp.zeros((pad,), jnp.int32)]).reshape(BP // CH, CH)
    lab_p = jnp.concatenate(
        [labels.reshape(-1), jnp.zeros((pad,), jnp.float32)]).reshape(BP // CH, CH)
    comb = _tc_pack(word_emb.T, context_emb.T)
    dots = _sc_dots(wrd_p, cx_p, comb)
    return _tc_loss(dots, lab_p)


# speedup vs baseline: 1.9055x; 1.9055x over previous
import functools

import jax
import jax.numpy as jnp
from jax import lax
from jax.experimental import pallas as pl
from jax.experimental.pallas import tpu as pltpu
from jax.experimental.pallas import tpu_sc as plsc

B = 130000
V = 1000000
H = 64
L = 16
NC, NS = 2, 16
NW = NC * NS
BP = 131072
BW = BP // NW
CH = 128
NCHUNK = BW // CH
NB = 2
NGRP = NCHUNK // NB
RW = BW // CH


def _sc_dots(wrd2d, cntxt2d, word_emb, context_emb):
    mesh = plsc.VectorSubcoreMesh(core_axis_name="c", subcore_axis_name="s")

    @functools.partial(
        pl.kernel,
        compiler_params=pltpu.CompilerParams(
            needs_layout_passes=False, use_tc_tiling_on_sc=True),
        out_type=jax.ShapeDtypeStruct((BP // CH, CH), jnp.float32),
        mesh=mesh,
        scratch_types=[
            pltpu.VMEM((NCHUNK, CH), jnp.int32),
            pltpu.VMEM((NCHUNK, CH), jnp.int32),
            [pltpu.VMEM((CH, H), jnp.float32)] * NB,
            [pltpu.VMEM((CH, H), jnp.float32)] * NB,
            pltpu.VMEM((NCHUNK, CH), jnp.float32),
            [pltpu.SemaphoreType.DMA] * NB,
        ],
    )
    def k(wrd_h, cx_h, wemb_h, cemb_h, out_h,
          widx, cidx, wrows, crows, dots, gsems):
        wid = lax.axis_index("s") * NC + lax.axis_index("c")
        r0 = pl.multiple_of(wid * RW, RW)

        pltpu.sync_copy(wrd_h.at[pl.ds(r0, RW)], widx)
        pltpu.sync_copy(cx_h.at[pl.ds(r0, RW)], cidx)

        def chunk_start(g, b):
            def sub(s, _):
                off = pl.multiple_of(s * L, L)
                ivw = widx[g, pl.ds(off, L)]
                ivc = cidx[g, pl.ds(off, L)]
                for j in range(L):
                    pltpu.make_async_copy(
                        wemb_h.at[pl.ds(ivw[j], 1)],
                        wrows[b].at[pl.ds(s * L + j, 1)], gsems[b]).start()
                    pltpu.make_async_copy(
                        cemb_h.at[pl.ds(ivc[j], 1)],
                        crows[b].at[pl.ds(s * L + j, 1)], gsems[b]).start()
                return 0
            lax.fori_loop(0, CH // L, sub, 0)

        def chunk_wait(b):
            pltpu.make_async_copy(
                wemb_h.at[pl.ds(0, CH)], wrows[b], gsems[b]).wait()
            pltpu.make_async_copy(
                cemb_h.at[pl.ds(0, CH)], crows[b], gsems[b]).wait()

        for b in range(NB):
            chunk_start(b, b)

        lanes = lax.iota(jnp.int32, L)
        lastlane = lanes == (L - 1)
        gsplat_cache = {}

        def dotitem(wr, cr, g, i):
            parts = []
            for q in range(4):
                sl = pl.ds(q * L, L)
                parts.append(wr[i, sl] * cr[i, sl])
            t = (parts[0] + parts[1]) + (parts[2] + parts[3])
            c = plsc.cumsum(t)
            plsc.store_scatter(
                dots, [gsplat_cache[0], jnp.full((L,), i, jnp.int32)],
                c, mask=lastlane)

        def compute(g, b):
            gsplat_cache[0] = jnp.full((L,), g, jnp.int32)

            def body(it, _):
                i = it * 2
                dotitem(wrows[b], crows[b], g, i)
                dotitem(wrows[b], crows[b], g, i + 1)
                return 0
            lax.fori_loop(0, CH // 2, body, 0)

        def grp_body(grp, _):
            for b in range(NB):
                g = grp * NB + b
                chunk_wait(b)
                compute(g, b)

                @pl.when(grp < NGRP - 1)
                def _():
                    chunk_start(g + NB, b)
            return 0

        lax.fori_loop(0, NGRP, grp_body, 0)

        pltpu.sync_copy(dots, out_h.at[pl.ds(r0, RW)])

    return k(wrd2d, cntxt2d, word_emb, context_emb)


def _tc_loss(dots2d, labels2d):
    def body(d_ref, l_ref, o_ref):
        x = d_ref[...] * l_ref[...]
        r = lax.broadcasted_iota(jnp.int32, x.shape, 0)
        c = lax.broadcasted_iota(jnp.int32, x.shape, 1)
        valid = (r * x.shape[1] + c) < B
        ls = jnp.where(valid, jax.nn.log_sigmoid(x), 0.0)
        o_ref[0, 0] = jnp.sum(ls) * (-1.0 / B)

    out = pl.pallas_call(
        body,
        out_shape=jax.ShapeDtypeStruct((1, 1), jnp.float32),
        out_specs=pl.BlockSpec(memory_space=pltpu.SMEM),
    )(dots2d, labels2d)
    return out[0, 0]


def kernel(wrd, cntxt, labels, word_emb, context_emb):
    pad = BP - B
    wrd_p = jnp.concatenate(
        [wrd.reshape(-1), jnp.zeros((pad,), jnp.int32)]).reshape(BP // CH, CH)
    cx_p = jnp.concatenate(
        [cntxt.reshape(-1), jnp.zeros((pad,), jnp.int32)]).reshape(BP // CH, CH)
    lab_p = jnp.concatenate(
        [labels.reshape(-1), jnp.zeros((pad,), jnp.float32)]).reshape(BP // CH, CH)
    dots = _sc_dots(wrd_p, cx_p, word_emb, context_emb)
    return _tc_loss(dots, lab_p)
